# baseline (speedup 1.0000x reference)
import functools

import jax
import jax.numpy as jnp
from jax import lax
from jax.experimental import pallas as pl
from jax.experimental.pallas import tpu as pltpu
from jax.experimental.pallas import tpu_sc as plsc

NUM_CLASSES = 1000
DIM = 64
BATCH = 16384
LANES = 16
SUBROWS = 8
NROWS = NUM_CLASSES * NUM_CLASSES

_info = plsc.get_sparse_core_info()
NUM_CORES = _info.num_cores
NUM_SUBCORES = _info.num_subcores
NW = NUM_CORES * NUM_SUBCORES
B_PER_W = BATCH // NW

TOP_ROWS = 393216
BOT_ROWS = NROWS - TOP_ROWS
T_BLK = 512


def _transpose_body(int_ref, out_ref):
    out_ref[...] = int_ref[...].T


def _tc_transpose(tablet):
    return pl.pallas_call(
        _transpose_body,
        grid=(TOP_ROWS // T_BLK,),
        in_specs=[pl.BlockSpec((DIM, T_BLK), lambda i: (0, i))],
        out_specs=pl.BlockSpec((T_BLK, DIM), lambda i: (i, 0)),
        out_shape=jax.ShapeDtypeStruct((TOP_ROWS, DIM), jnp.float32),
    )(tablet)


def _freq_bias_body(sbj_hbm, obj_hbm, top_hbm, bot_hbm, out_hbm,
                    sbj_v, obj_v, outb_v, sem):
    wid = lax.axis_index("s") * NUM_CORES + lax.axis_index("c")
    base = wid * B_PER_W
    pltpu.sync_copy(sbj_hbm.at[pl.ds(base, B_PER_W)], sbj_v)
    pltpu.sync_copy(obj_hbm.at[pl.ds(base, B_PER_W)], obj_v)

    def group_body(g, _):
        s = sbj_v[pl.ds(g * LANES, LANES)]
        o = obj_v[pl.ds(g * LANES, LANES)]
        f_vec = s * NUM_CLASSES + o
        for l in range(LANES):
            e = g * LANES + l
            f = f_vec[l]
            dst = outb_v.at[lax.shift_right_logical(e, 3),
                            lax.bitwise_and(e, 7)]

            def _from_top(f=f, dst=dst):
                pltpu.async_copy(top_hbm.at[f], dst, sem)

            def _from_bot(f=f, dst=dst):
                fb = f - TOP_ROWS
                pltpu.async_copy(
                    bot_hbm.at[lax.shift_right_logical(fb, 3),
                               lax.bitwise_and(fb, 7)],
                    dst, sem)

            lax.cond(f < TOP_ROWS, _from_top, _from_bot)
        return _

    lax.fori_loop(0, B_PER_W // LANES, group_body, None)

    pltpu.make_async_copy(bot_hbm.at[pl.ds(0, B_PER_W // SUBROWS)],
                          outb_v, sem).wait()

    pltpu.sync_copy(outb_v,
                    out_hbm.at[pl.ds(wid * (B_PER_W // SUBROWS),
                                     B_PER_W // SUBROWS)])


def kernel(sbj_labels, obj_labels, node_baseline):
    mesh = plsc.VectorSubcoreMesh(core_axis_name="c", subcore_axis_name="s")
    k = pl.kernel(
        _freq_bias_body,
        mesh=mesh,
        compiler_params=pltpu.CompilerParams(use_tc_tiling_on_sc=True),
        out_type=jax.ShapeDtypeStruct((BATCH // SUBROWS, SUBROWS, DIM),
                                      jnp.float32),
        scratch_types=[
            pltpu.VMEM((B_PER_W,), jnp.int32),
            pltpu.VMEM((B_PER_W,), jnp.int32),
            pltpu.VMEM((B_PER_W // SUBROWS, SUBROWS, DIM), jnp.float32),
            pltpu.SemaphoreType.DMA,
        ],
    )
    top = _tc_transpose(node_baseline.T)
    bot = node_baseline[TOP_ROWS:].reshape(BOT_ROWS // SUBROWS, SUBROWS, DIM)
    out3 = k(sbj_labels.astype(jnp.int32), obj_labels.astype(jnp.int32),
             top, bot)
    return out3.reshape(BATCH, DIM)


# speedup vs baseline: 1.2452x; 1.2452x over previous
import jax
import jax.numpy as jnp
from jax import lax
from jax.experimental import pallas as pl
from jax.experimental.pallas import tpu as pltpu
from jax.experimental.pallas import tpu_sc as plsc

NUM_CLASSES = 1000
DIM = 64
BATCH = 16384
LANES = 16
SUBROWS = 8
NROWS = NUM_CLASSES * NUM_CLASSES

_info = plsc.get_sparse_core_info()
NUM_CORES = _info.num_cores
NUM_SUBCORES = _info.num_subcores
NW = NUM_CORES * NUM_SUBCORES
B_PER_W = BATCH // NW

T_BLK = 2048


def _transpose_body(int_ref, out_ref):
    eye = (lax.broadcasted_iota(jnp.int32, (DIM, DIM), 0)
           == lax.broadcasted_iota(jnp.int32, (DIM, DIM), 1)
           ).astype(jnp.float32)
    out_ref[...] = lax.dot_general(
        int_ref[...], eye, (((0,), (0,)), ((), ())),
        preferred_element_type=jnp.float32)


def _tc_transpose(tablet):
    return pl.pallas_call(
        _transpose_body,
        grid=(pl.cdiv(NROWS, T_BLK),),
        in_specs=[pl.BlockSpec((DIM, T_BLK), lambda i: (0, i))],
        out_specs=pl.BlockSpec((T_BLK, DIM), lambda i: (i, 0)),
        out_shape=jax.ShapeDtypeStruct((NROWS, DIM), jnp.float32),
    )(tablet)


def _freq_bias_body(sbj_hbm, obj_hbm, table_hbm, out_hbm,
                    sbj_v, obj_v, outb_v, sem):
    wid = lax.axis_index("s") * NUM_CORES + lax.axis_index("c")
    base = wid * B_PER_W
    pltpu.sync_copy(sbj_hbm.at[pl.ds(base, B_PER_W)], sbj_v)
    pltpu.sync_copy(obj_hbm.at[pl.ds(base, B_PER_W)], obj_v)

    def group_body(g, _):
        s = sbj_v[pl.ds(g * LANES, LANES)]
        o = obj_v[pl.ds(g * LANES, LANES)]
        f_vec = s * NUM_CLASSES + o
        for l in range(LANES):
            e = g * LANES + l
            pltpu.async_copy(
                table_hbm.at[f_vec[l]],
                outb_v.at[lax.shift_right_logical(e, 3),
                          lax.bitwise_and(e, 7)],
                sem)
        return _

    lax.fori_loop(0, B_PER_W // LANES, group_body, None)

    pltpu.make_async_copy(out_hbm.at[pl.ds(0, B_PER_W // SUBROWS)],
                          outb_v, sem).wait()

    pltpu.sync_copy(outb_v,
                    out_hbm.at[pl.ds(wid * (B_PER_W // SUBROWS),
                                     B_PER_W // SUBROWS)])


def kernel(sbj_labels, obj_labels, node_baseline):
    mesh = plsc.VectorSubcoreMesh(core_axis_name="c", subcore_axis_name="s")
    k = pl.kernel(
        _freq_bias_body,
        mesh=mesh,
        compiler_params=pltpu.CompilerParams(use_tc_tiling_on_sc=True),
        out_type=jax.ShapeDtypeStruct((BATCH // SUBROWS, SUBROWS, DIM),
                                      jnp.float32),
        scratch_types=[
            pltpu.VMEM((B_PER_W,), jnp.int32),
            pltpu.VMEM((B_PER_W,), jnp.int32),
            pltpu.VMEM((B_PER_W // SUBROWS, SUBROWS, DIM), jnp.float32),
            pltpu.SemaphoreType.DMA,
        ],
    )
    table_rm = _tc_transpose(node_baseline.T)
    out3 = k(sbj_labels.astype(jnp.int32), obj_labels.astype(jnp.int32),
             table_rm)
    return out3.reshape(BATCH, DIM)


# speedup vs baseline: 2.6015x; 2.0892x over previous
import jax
import jax.numpy as jnp
from jax import lax
from jax.experimental import pallas as pl
from jax.experimental.pallas import tpu as pltpu
from jax.experimental.pallas import tpu_sc as plsc

NUM_CLASSES = 1000
DIM = 64
BATCH = 16384
LANES = 16
SUBROWS = 8

_info = plsc.get_sparse_core_info()
NUM_CORES = _info.num_cores
NUM_SUBCORES = _info.num_subcores
NW = NUM_CORES * NUM_SUBCORES
B_PER_W = BATCH // NW
NTILE = 125000


def _freq_bias_body(sbj_hbm, obj_hbm, table_hbm, out_hbm,
                    sbj_v, obj_v, outb_v, sem):
    wid = lax.axis_index("s") * NUM_CORES + lax.axis_index("c")
    base = wid * B_PER_W
    pltpu.sync_copy(sbj_hbm.at[pl.ds(base, B_PER_W)], sbj_v)
    pltpu.sync_copy(obj_hbm.at[pl.ds(base, B_PER_W)], obj_v)

    def group_body(g, _):
        s = sbj_v[pl.ds(g * LANES, LANES)]
        o = obj_v[pl.ds(g * LANES, LANES)]
        f_vec = s * NUM_CLASSES + o
        t_vec = lax.shift_right_logical(f_vec, 3)
        r_vec = lax.bitwise_and(f_vec, 7)
        for l in range(LANES):
            e = g * LANES + l
            pltpu.async_copy(
                table_hbm.at[t_vec[l], r_vec[l]],
                outb_v.at[lax.shift_right_logical(e, 3),
                          lax.bitwise_and(e, 7)],
                sem)
        return _

    lax.fori_loop(0, B_PER_W // LANES, group_body, None)

    pltpu.make_async_copy(table_hbm.at[pl.ds(0, B_PER_W // SUBROWS)],
                          outb_v, sem).wait()

    pltpu.sync_copy(outb_v,
                    out_hbm.at[pl.ds(wid * (B_PER_W // SUBROWS),
                                     B_PER_W // SUBROWS)])


def kernel(sbj_labels, obj_labels, node_baseline):
    mesh = plsc.VectorSubcoreMesh(core_axis_name="c", subcore_axis_name="s")
    k = pl.kernel(
        _freq_bias_body,
        mesh=mesh,
        compiler_params=pltpu.CompilerParams(use_tc_tiling_on_sc=True),
        out_type=jax.ShapeDtypeStruct((BATCH // SUBROWS, SUBROWS, DIM),
                                      jnp.float32),
        scratch_types=[
            pltpu.VMEM((B_PER_W,), jnp.int32),
            pltpu.VMEM((B_PER_W,), jnp.int32),
            pltpu.VMEM((B_PER_W // SUBROWS, SUBROWS, DIM), jnp.float32),
            pltpu.SemaphoreType.DMA,
        ],
    )
    table3 = node_baseline.reshape(NTILE, SUBROWS, DIM)
    out3 = k(sbj_labels.astype(jnp.int32), obj_labels.astype(jnp.int32),
             table3)
    return out3.reshape(BATCH, DIM)
